# Initial kernel scaffold; baseline (speedup 1.0000x reference)
import jax
import jax.numpy as jnp
from jax.experimental import pallas as pl


def kernel(feat_0, feat_1, feat_2, feat_3, node_table, emb_0, emb_1, emb_2, emb_3, W1, b1, W2, b2):
    raise NotImplementedError("write your pallas kernel here")


# speedup vs baseline: 2.0745x; 2.0745x over previous
import functools

import jax
import jax.numpy as jnp
from jax import lax
from jax.experimental import pallas as pl
from jax.experimental.pallas import tpu as pltpu
from jax.experimental.pallas import tpu_sc as plsc

B = 16384
D = 64
F = 4

_NC = 2
_NS = 16
NW = _NC * _NS
BPW = B // NW
CH = 128
NCH = BPW // CH

def _take(x, idx):
    dnums = lax.GatherDimensionNumbers(
        offset_dims=(), collapsed_slice_dims=(0,), start_index_map=(0,))
    return lax.gather(x, idx[:, None], dnums, (1,),
                      mode=lax.GatherScatterMode.PROMISE_IN_BOUNDS)


def _sc_body(f0, f1, f2, f3, node, e0, e1, e2, e3, w1, b1p, w2, b2p,
             out, wv, i0, i1, i2, i3, nbuf, g0, g1, g2, g3, sem):
    wid = lax.axis_index("s") * _NC + lax.axis_index("c")
    base = wid * BPW

    pltpu.sync_copy(w1, wv.at[0])
    pltpu.sync_copy(b1p, wv.at[1])
    pltpu.sync_copy(w2, wv.at[2])
    pltpu.sync_copy(b2p, wv.at[3])
    lane = lax.iota(jnp.int32, 16)
    ridx = lax.convert_element_type(lax.shift_right_logical(lane, 2), jnp.float32)
    t = wv[0, :] * ridx
    u = t + _take(t, lane ^ 8)
    v = u + _take(u, lane ^ 4)
    h = jnp.maximum(v + wv[1, :], 0.0)
    h = jnp.where(lane < 4, h, 0.0)
    hb = _take(h, lax.shift_right_logical(lane, 2))
    t2 = hb * wv[2, :]
    u2 = t2 + _take(t2, lane ^ 8)
    v2 = u2 + _take(u2, lane ^ 4)
    lg = jnp.where(lane < 4, v2 + wv[3, :], -1e30)
    m = lg
    for sh in (8, 4, 2, 1):
        m = jnp.maximum(m, _take(m, lane ^ sh))
    e = jnp.exp(lg - m)
    s = e
    for sh in (8, 4, 2, 1):
        s = s + _take(s, lane ^ sh)
    att = e / s
    a0 = _take(att, jnp.full((16,), 0, jnp.int32))
    a1 = _take(att, jnp.full((16,), 1, jnp.int32))
    a2 = _take(att, jnp.full((16,), 2, jnp.int32))
    a3 = _take(att, jnp.full((16,), 3, jnp.int32))

    for c in range(NCH):
        r0 = base + c * CH
        pltpu.sync_copy(f0.at[pl.ds(r0, CH)], i0.at[c])
        pltpu.sync_copy(f1.at[pl.ds(r0, CH)], i1.at[c])
        pltpu.sync_copy(f2.at[pl.ds(r0, CH)], i2.at[c])
        pltpu.sync_copy(f3.at[pl.ds(r0, CH)], i3.at[c])
        cps = [
            pltpu.async_copy(node.at[i0.at[c]], nbuf, sem),
            pltpu.async_copy(e0.at[i0.at[c]], g0, sem),
            pltpu.async_copy(e1.at[i1.at[c]], g1, sem),
            pltpu.async_copy(e2.at[i2.at[c]], g2, sem),
            pltpu.async_copy(e3.at[i3.at[c]], g3, sem),
        ]
        for cp in cps:
            cp.wait()

        def body(r, carry):
            for j in range(D // 16):
                sl = pl.ds(16 * j, 16)
                nbuf[r, sl] = (nbuf[r, sl] + a0 * g0[r, sl] + a1 * g1[r, sl]
                               + a2 * g2[r, sl] + a3 * g3[r, sl])
            return carry

        lax.fori_loop(0, CH, body, 0)
        pltpu.sync_copy(nbuf, out.at[pl.ds(r0, CH)])


_eges_sc_cache = []


def _eges_sc():
    if not _eges_sc_cache:
        _eges_sc_cache.append(functools.partial(
            pl.kernel,
            mesh=plsc.VectorSubcoreMesh(core_axis_name="c",
                                        subcore_axis_name="s"),
            compiler_params=pltpu.CompilerParams(use_tc_tiling_on_sc=False),
            out_type=jax.ShapeDtypeStruct((B, D), jnp.float32),
            scratch_types=[
                pltpu.VMEM((4, 16), jnp.float32),
                pltpu.VMEM((NCH, CH), jnp.int32),
                pltpu.VMEM((NCH, CH), jnp.int32),
                pltpu.VMEM((NCH, CH), jnp.int32),
                pltpu.VMEM((NCH, CH), jnp.int32),
                pltpu.VMEM((CH, D), jnp.float32),
                pltpu.VMEM((CH, D), jnp.float32),
                pltpu.VMEM((CH, D), jnp.float32),
                pltpu.VMEM((CH, D), jnp.float32),
                pltpu.VMEM((CH, D), jnp.float32),
                pltpu.SemaphoreType.DMA,
            ],
        )(_sc_body))
    return _eges_sc_cache[0]


@jax.jit
def kernel(feat_0, feat_1, feat_2, feat_3, node_table,
           emb_0, emb_1, emb_2, emb_3, W1, b1, W2, b2):
    w1f = W1.reshape(16)
    w2f = W2.reshape(16)
    b1p = jnp.pad(b1, (0, 12))
    b2p = jnp.pad(b2, (0, 12))
    return _eges_sc()(feat_0, feat_1, feat_2, feat_3, node_table,
                      emb_0, emb_1, emb_2, emb_3, w1f, b1p, w2f, b2p)
